# baseline (speedup 1.0000x reference)
import jax
import jax.numpy as jnp
from jax import lax
from jax.experimental import pallas as pl
from jax.experimental.pallas import tpu as pltpu
from jax.experimental.pallas import tpu_sc as plsc

_BATCH = 4096
_HIST = 50
_D = 64
_B = _BATCH * _HIST
_R = _B // 2
_NC = 2
_NS = 16
_NW = _NC * _NS
_RPW = _R // _NW
_GR = 640
_RG = _RPW // _GR
_NG = 2 * _RG


def _emb_body(idx_hbm, table_hbm, out_hbm, idx_v, rows_a, rows_b,
              isem, gsem_a, gsem_b, wsem_a, wsem_b):
  wid = lax.axis_index("s") * _NC + lax.axis_index("c")
  rows = (rows_a, rows_b)
  gsem = (gsem_a, gsem_b)
  wsem = (wsem_a, wsem_b)

  r0 = wid * _RPW
  pltpu.sync_copy(idx_hbm.at[pl.ds(r0, _RPW)], idx_v.at[pl.ds(0, _RPW)])
  pltpu.sync_copy(idx_hbm.at[pl.ds(_R + r0, _RPW)],
                  idx_v.at[pl.ds(_RPW, _RPW)])

  def start_gather(g):
    b = g % 2
    rg, par = g // 2, g % 2
    return pltpu.async_copy(
        table_hbm.at[idx_v.at[pl.ds(par * _RPW + rg * _GR, _GR)]],
        rows[b], gsem[b])

  def start_write(g):
    b = g % 2
    rg, par = g // 2, g % 2
    return pltpu.async_copy(
        rows[b],
        out_hbm.at[pl.ds(r0 + rg * _GR, _GR), pl.ds(par * _D, _D)],
        wsem[b])

  gathers = [None] * _NG
  writes = [None] * _NG
  gathers[0] = start_gather(0)
  for g in range(_NG):
    if g + 1 < _NG:
      if g >= 1:
        writes[g - 1].wait()
      gathers[g + 1] = start_gather(g + 1)
    gathers[g].wait()
    writes[g] = start_write(g)
  writes[_NG - 2].wait()
  writes[_NG - 1].wait()


_emb_call = pl.kernel(
    _emb_body,
    out_type=jax.ShapeDtypeStruct((_R, 2 * _D), jnp.float32),
    mesh=plsc.VectorSubcoreMesh(core_axis_name="c", subcore_axis_name="s"),
    scratch_types=[
        pltpu.VMEM((2 * _RPW,), jnp.int32),
        pltpu.VMEM((_GR, _D), jnp.float32),
        pltpu.VMEM((_GR, _D), jnp.float32),
        pltpu.SemaphoreType.DMA,
        pltpu.SemaphoreType.DMA,
        pltpu.SemaphoreType.DMA,
        pltpu.SemaphoreType.DMA,
        pltpu.SemaphoreType.DMA,
    ],
    compiler_params=pltpu.CompilerParams(use_tc_tiling_on_sc=False),
)


@jax.jit
def kernel(input, weights):
  idx_eo = input.astype(jnp.int32).reshape(_R, 2).T.reshape(_B)
  packed = _emb_call(idx_eo, weights)
  return packed.reshape(_BATCH, _HIST, _D)
